# Initial kernel scaffold; baseline (speedup 1.0000x reference)
import jax
import jax.numpy as jnp
from jax.experimental import pallas as pl


def kernel(features, edge_index, cheb_W, cheb_b, mlp_W0, mlp_b0, mlp_W1, mlp_b1, mlp_W2, mlp_b2):
    raise NotImplementedError("write your pallas kernel here")


# speedup vs baseline: 5.6850x; 5.6850x over previous
import functools

import jax
import jax.numpy as jnp
from jax import lax
from jax.experimental import pallas as pl
from jax.experimental.pallas import tpu as pltpu
from jax.experimental.pallas import tpu_sc as plsc

_N = 10000
_E = 320000
_D = 128
_NC = 2
_NS = 16
_NW = _NC * _NS
_EPT = _E // _NW
_C = 80
_NCH = _EPT // _C
_NP = 10240
_RPT = _NP // _NS
_ZR = 128

_BR = 400
_G = _N // _BR


def _sc_mesh():
    return plsc.VectorSubcoreMesh(core_axis_name="c", subcore_axis_name="s")


@functools.partial(
    pl.kernel,
    out_type=jax.ShapeDtypeStruct((_NC, _NP, _D), jnp.float32),
    mesh=_sc_mesh(),
    scratch_types=[
        pltpu.VMEM((_C,), jnp.int32),
        pltpu.VMEM((_C,), jnp.int32),
        pltpu.VMEM((_C, _D), jnp.float32),
        pltpu.VMEM((_ZR, _D), jnp.float32),
        pltpu.VMEM_SHARED((_NP, _D), jnp.float32),
        pltpu.SemaphoreType.DMA,
    ],
)
def _sc_segsum(g_hbm, src_hbm, dst_hbm, out_hbm,
               src_v, dst_v, rows_v, zero_v, acc_sh, sem):
    c = lax.axis_index("c")
    s = lax.axis_index("s")

    def _zrow(i, carry):
        for j in range(_D // 16):
            zero_v[i, pl.ds(j * 16, 16)] = jnp.zeros((16,), jnp.float32)
        return carry
    lax.fori_loop(0, _ZR, _zrow, 0)

    row0 = s * _RPT
    for t in range(_RPT // _ZR):
        pltpu.sync_copy(zero_v, acc_sh.at[pl.ds(row0 + t * _ZR, _ZR)])
    plsc.subcore_barrier()

    base_e = (c * _NS + s) * _EPT

    def _chunk(i, carry):
        off = base_e + i * _C
        pltpu.sync_copy(src_hbm.at[pl.ds(off, _C)], src_v)
        pltpu.sync_copy(dst_hbm.at[pl.ds(off, _C)], dst_v)
        pltpu.async_copy(g_hbm.at[src_v], rows_v, sem).wait()
        pltpu.sync_copy(rows_v, acc_sh.at[dst_v], add=True)
        return carry
    lax.fori_loop(0, _NCH, _chunk, 0)

    plsc.subcore_barrier()
    pltpu.sync_copy(acc_sh.at[pl.ds(row0, _RPT)],
                    out_hbm.at[c, pl.ds(row0, _RPT)])


def _tc_prep(degp, feats):
    def body(degp_ref, f_ref, inv_ref, g_ref):
        deg = degp_ref[0, :, 0:1] + degp_ref[1, :, 0:1]
        inv = lax.rsqrt(jnp.maximum(deg, 1.0))
        inv_ref[...] = jnp.broadcast_to(inv, (_BR, _D))
        g_ref[...] = f_ref[...] * inv

    return pl.pallas_call(
        body,
        grid=(_G,),
        in_specs=[
            pl.BlockSpec((_NC, _BR, _D), lambda i: (0, i, 0)),
            pl.BlockSpec((_BR, _D), lambda i: (i, 0)),
        ],
        out_specs=[
            pl.BlockSpec((_BR, _D), lambda i: (i, 0)),
            pl.BlockSpec((_BR, _D), lambda i: (i, 0)),
        ],
        out_shape=[
            jax.ShapeDtypeStruct((_N, _D), jnp.float32),
            jax.ShapeDtypeStruct((_N, _D), jnp.float32),
        ],
    )(degp, feats)


def _tc_layer1(h, s_part, inv, w01):
    def body(h_ref, s_ref, inv_ref, w_ref, out_ref, g2_ref):
        ssum = s_ref[0] + s_ref[1]
        inv_v = inv_ref[...]
        x1 = -(inv_v * ssum)
        out_ref[...] = (
            jnp.dot(h_ref[...], w_ref[0], preferred_element_type=jnp.float32)
            + jnp.dot(x1, w_ref[1], preferred_element_type=jnp.float32))
        g2_ref[...] = inv_v * x1

    return pl.pallas_call(
        body,
        grid=(_G,),
        in_specs=[
            pl.BlockSpec((_BR, _D), lambda i: (i, 0)),
            pl.BlockSpec((_NC, _BR, _D), lambda i: (0, i, 0)),
            pl.BlockSpec((_BR, _D), lambda i: (i, 0)),
            pl.BlockSpec((2, _D, _D), lambda i: (0, 0, 0)),
        ],
        out_specs=[
            pl.BlockSpec((_BR, _D), lambda i: (i, 0)),
            pl.BlockSpec((_BR, _D), lambda i: (i, 0)),
        ],
        out_shape=[
            jax.ShapeDtypeStruct((_N, _D), jnp.float32),
            jax.ShapeDtypeStruct((_N, _D), jnp.float32),
        ],
    )(h, s_part, inv, w01)


def _tc_layer2(out_acc, h, s_part, inv, w2, b):
    def body(o_ref, h_ref, s_ref, inv_ref, w_ref, b_ref, hn_ref, gn_ref):
        ssum = s_ref[0] + s_ref[1]
        inv_v = inv_ref[...]
        x2 = -2.0 * inv_v * ssum - h_ref[...]
        hn = jnp.maximum(
            o_ref[...]
            + jnp.dot(x2, w_ref[...], preferred_element_type=jnp.float32)
            + b_ref[...], 0.0)
        hn_ref[...] = hn
        gn_ref[...] = inv_v * hn

    return pl.pallas_call(
        body,
        grid=(_G,),
        in_specs=[
            pl.BlockSpec((_BR, _D), lambda i: (i, 0)),
            pl.BlockSpec((_BR, _D), lambda i: (i, 0)),
            pl.BlockSpec((_NC, _BR, _D), lambda i: (0, i, 0)),
            pl.BlockSpec((_BR, _D), lambda i: (i, 0)),
            pl.BlockSpec((_D, _D), lambda i: (0, 0)),
            pl.BlockSpec((1, _D), lambda i: (0, 0)),
        ],
        out_specs=[
            pl.BlockSpec((_BR, _D), lambda i: (i, 0)),
            pl.BlockSpec((_BR, _D), lambda i: (i, 0)),
        ],
        out_shape=[
            jax.ShapeDtypeStruct((_N, _D), jnp.float32),
            jax.ShapeDtypeStruct((_N, _D), jnp.float32),
        ],
    )(out_acc, h, s_part, inv, w2, b)


def _tc_head(h, w0, b0, w1, b1, w2, b2):
    def body(h_ref, w0_ref, b0_ref, w1_ref, b1_ref, w2_ref, b2_ref,
             out_ref, acc_ref):
        i = pl.program_id(0)

        @pl.when(i == 0)
        def _():
            acc_ref[...] = jnp.zeros_like(acc_ref)

        acc_ref[...] += jnp.sum(h_ref[...], axis=0, keepdims=True)

        @pl.when(i == _G - 1)
        def _():
            hg = acc_ref[...] * (1.0 / _N)
            x = jnp.maximum(
                jnp.dot(hg, w0_ref[...], preferred_element_type=jnp.float32)
                + b0_ref[...], 0.0)
            x = jnp.maximum(
                jnp.dot(x, w1_ref[...], preferred_element_type=jnp.float32)
                + b1_ref[...], 0.0)
            out_ref[...] = (
                jnp.dot(x, w2_ref[...], preferred_element_type=jnp.float32)
                + b2_ref[...])

    return pl.pallas_call(
        body,
        grid=(_G,),
        in_specs=[
            pl.BlockSpec((_BR, _D), lambda i: (i, 0)),
            pl.BlockSpec((_D, _D // 2), lambda i: (0, 0)),
            pl.BlockSpec((1, _D // 2), lambda i: (0, 0)),
            pl.BlockSpec((_D // 2, _D // 4), lambda i: (0, 0)),
            pl.BlockSpec((1, _D // 4), lambda i: (0, 0)),
            pl.BlockSpec((_D // 4, 40), lambda i: (0, 0)),
            pl.BlockSpec((1, 40), lambda i: (0, 0)),
        ],
        out_specs=pl.BlockSpec((1, 40), lambda i: (0, 0)),
        out_shape=jax.ShapeDtypeStruct((1, 40), jnp.float32),
        scratch_shapes=[pltpu.VMEM((1, _D), jnp.float32)],
    )(h, w0, b0, w1, b1, w2, b2)


def kernel(features, edge_index, cheb_W, cheb_b,
           mlp_W0, mlp_b0, mlp_W1, mlp_b1, mlp_W2, mlp_b2):
    src = edge_index[0]
    dst = edge_index[1]

    ones_tab = jnp.ones((_N, _D), jnp.float32)
    degp = _sc_segsum(ones_tab, src, dst)
    inv, g = _tc_prep(degp, features)

    h = features
    n_layers = cheb_W.shape[0]
    for layer in range(n_layers):
        s1 = _sc_segsum(g, src, dst)
        out_acc, g2 = _tc_layer1(h, s1, inv, cheb_W[layer, 0:2])
        s2 = _sc_segsum(g2, src, dst)
        h, g = _tc_layer2(out_acc, h, s2, inv, cheb_W[layer, 2],
                          cheb_b[layer].reshape(1, -1))

    return _tc_head(h, mlp_W0, mlp_b0.reshape(1, -1),
                    mlp_W1, mlp_b1.reshape(1, -1),
                    mlp_W2, mlp_b2.reshape(1, -1))


# speedup vs baseline: 14.0381x; 2.4693x over previous
import functools

import jax
import jax.numpy as jnp
from jax import lax
from jax.experimental import pallas as pl
from jax.experimental.pallas import tpu as pltpu
from jax.experimental.pallas import tpu_sc as plsc

_N = 10000
_E = 320000
_D = 128
_NC = 2
_NS = 16
_NW = _NC * _NS
_EPT = _E // _NW
_C = 80
_NCH = _EPT // _C
_NP = 10240
_RPT = _NP // _NS
_ZR = 32
_NZ = _RPT // _ZR

_BR = 400
_G = _N // _BR


def _sc_mesh():
    return plsc.VectorSubcoreMesh(core_axis_name="c", subcore_axis_name="s")


@functools.partial(
    pl.kernel,
    out_type=jax.ShapeDtypeStruct((_NC, _NP, _D), jnp.float32),
    mesh=_sc_mesh(),
    scratch_types=[
        pltpu.VMEM((_C,), jnp.int32),
        pltpu.VMEM((_C,), jnp.int32),
        pltpu.VMEM((_C,), jnp.int32),
        pltpu.VMEM((_C,), jnp.int32),
        pltpu.VMEM((8, _C), jnp.int32),
        pltpu.VMEM((8, _C), jnp.int32),
        pltpu.VMEM((8, _C), jnp.int32),
        pltpu.VMEM((8, _C), jnp.int32),
        pltpu.VMEM((_C, _D), jnp.float32),
        pltpu.VMEM((_C, _D), jnp.float32),
        pltpu.VMEM((_ZR, _D), jnp.float32),
        pltpu.VMEM_SHARED((_NP, _D), jnp.float32),
        pltpu.SemaphoreType.DMA,
        pltpu.SemaphoreType.DMA,
        pltpu.SemaphoreType.DMA,
        pltpu.SemaphoreType.DMA,
        pltpu.SemaphoreType.DMA,
        pltpu.SemaphoreType.DMA,
        pltpu.SemaphoreType.DMA,
        pltpu.SemaphoreType.DMA,
        pltpu.SemaphoreType.DMA,
        pltpu.SemaphoreType.DMA,
        pltpu.SemaphoreType.DMA,
        pltpu.SemaphoreType.DMA,
        pltpu.SemaphoreType.DMA,
    ],
)
def _sc_segsum(g_hbm, src_hbm, dst_hbm, out_hbm,
               sb0, sb1, sb2, sb3, db0, db1, db2, db3,
               buf0, buf1, zero_v, acc_sh,
               ss0, ss1, ss2, ss3, ds0, ds1, ds2, ds3,
               gs0, gs1, as0, as1, zs):
    c = lax.axis_index("c")
    s = lax.axis_index("s")
    base_e = (c * _NS + s) * _EPT

    sbs = (sb0, sb1, sb2, sb3)
    dbs = (db0, db1, db2, db3)
    bufs = (buf0, buf1)
    ssems = (ss0, ss1, ss2, ss3)
    dsems = (ds0, ds1, ds2, ds3)
    gsems = (gs0, gs1)
    asems = (as0, as1)

    def _idx_load(i, q):
        off = base_e + i * _C
        pltpu.async_copy(src_hbm.at[pl.ds(off, _C)], sbs[q], ssems[q])
        pltpu.async_copy(dst_hbm.at[pl.ds(off, _C)], dbs[q].at[0], dsems[q])

    def _idx_wait(q):
        pltpu.make_async_copy(src_hbm.at[pl.ds(0, _C)], sbs[q],
                              ssems[q]).wait()
        pltpu.make_async_copy(dst_hbm.at[pl.ds(0, _C)], dbs[q].at[0],
                              dsems[q]).wait()

    def _gather(b, q):
        pltpu.async_copy(g_hbm.at[sbs[q]], bufs[b], gsems[b])

    def _gather_wait(b):
        pltpu.make_async_copy(g_hbm.at[pl.ds(0, _C)], bufs[b],
                              gsems[b]).wait()

    def _scat(b, q):
        pltpu.async_copy(bufs[b], acc_sh.at[dbs[q].at[0]], asems[b],
                         add=True)

    def _scat_wait(b):
        pltpu.make_async_copy(bufs[b], acc_sh.at[pl.ds(0, _C)],
                              asems[b]).wait()

    _idx_load(0, 0)
    _idx_load(1, 1)
    _idx_load(2, 2)

    def _zrow(i, carry):
        for j in range(_D // 16):
            zero_v[i, pl.ds(j * 16, 16)] = jnp.zeros((16,), jnp.float32)
        return carry
    lax.fori_loop(0, _ZR, _zrow, 0)

    row0 = s * _RPT
    for t in range(_NZ):
        pltpu.async_copy(zero_v, acc_sh.at[pl.ds(row0 + t * _ZR, _ZR)], zs)

    _idx_wait(0)
    _gather(0, 0)

    for t in range(_NZ):
        pltpu.make_async_copy(zero_v, acc_sh.at[pl.ds(row0, _ZR)], zs).wait()
    plsc.subcore_barrier()

    def _chunk(i, b, q, load):
        _scat_wait(b)
        if load:
            _idx_load(i + 2, (q + 2) % 4)
        _idx_wait(q)
        _gather(b, q)
        _gather_wait(1 - b)
        _scat(1 - b, (q + 3) % 4)

    _idx_load(3, 3)
    _idx_wait(1)
    _gather(1, 1)
    _gather_wait(0)
    _scat(0, 0)

    def _quad(g, carry):
        i0 = 4 * g + 2
        _chunk(i0, 0, 2, True)
        _chunk(i0 + 1, 1, 3, True)
        _chunk(i0 + 2, 0, 0, True)
        _chunk(i0 + 3, 1, 1, True)
        return carry
    lax.fori_loop(0, (_NCH - 5) // 4, _quad, 0)

    _chunk(_NCH - 3, 0, 2, True)
    _chunk(_NCH - 2, 1, 3, False)
    _chunk(_NCH - 1, 0, 0, False)

    _gather_wait(0)
    _scat(0, 0)
    _scat_wait(1)
    _scat_wait(0)

    plsc.subcore_barrier()
    pltpu.sync_copy(acc_sh.at[pl.ds(row0, _RPT)],
                    out_hbm.at[c, pl.ds(row0, _RPT)])


@functools.partial(
    pl.kernel,
    out_type=jax.ShapeDtypeStruct((_NC, _NP, _D), jnp.float32),
    mesh=_sc_mesh(),
    scratch_types=[
        pltpu.VMEM((8, _C), jnp.int32),
        pltpu.VMEM((8, _C), jnp.int32),
        pltpu.VMEM((8, _C), jnp.int32),
        pltpu.VMEM((8, _C), jnp.int32),
        pltpu.VMEM((_C, _D), jnp.float32),
        pltpu.VMEM((_ZR, _D), jnp.float32),
        pltpu.VMEM_SHARED((_NP, _D), jnp.float32),
        pltpu.SemaphoreType.DMA,
        pltpu.SemaphoreType.DMA,
        pltpu.SemaphoreType.DMA,
        pltpu.SemaphoreType.DMA,
        pltpu.SemaphoreType.DMA,
        pltpu.SemaphoreType.DMA,
        pltpu.SemaphoreType.DMA,
    ],
)
def _sc_degree(dst_hbm, out_hbm,
               cb0, cb1, cb2, cb3, ones_v, zero_v, acc_sh,
               is0, is1, is2, is3, as0, as1, zs):
    c = lax.axis_index("c")
    s = lax.axis_index("s")
    base_e = (c * _NS + s) * _EPT

    cbs = (cb0, cb1, cb2, cb3)
    isems = (is0, is1, is2, is3)
    asems = (as0, as1)

    def _idx_load(i, q):
        off = base_e + i * _C
        pltpu.async_copy(dst_hbm.at[pl.ds(off, _C)], cbs[q].at[0], isems[q])

    def _idx_wait(q):
        pltpu.make_async_copy(dst_hbm.at[pl.ds(0, _C)], cbs[q].at[0],
                              isems[q]).wait()

    def _scat(b, q):
        pltpu.async_copy(ones_v, acc_sh.at[cbs[q].at[0]], asems[b],
                         add=True)

    def _scat_wait(b):
        pltpu.make_async_copy(ones_v, acc_sh.at[pl.ds(0, _C)],
                              asems[b]).wait()

    _idx_load(0, 0)
    _idx_load(1, 1)
    _idx_load(2, 2)
    _idx_load(3, 3)

    def _zrow(i, carry):
        for j in range(_D // 16):
            zero_v[i, pl.ds(j * 16, 16)] = jnp.zeros((16,), jnp.float32)
        return carry
    lax.fori_loop(0, _ZR, _zrow, 0)

    def _orow(i, carry):
        for j in range(_D // 16):
            ones_v[i, pl.ds(j * 16, 16)] = jnp.ones((16,), jnp.float32)
        return carry
    lax.fori_loop(0, _C, _orow, 0)

    row0 = s * _RPT
    for t in range(_NZ):
        pltpu.async_copy(zero_v, acc_sh.at[pl.ds(row0 + t * _ZR, _ZR)], zs)
    for t in range(_NZ):
        pltpu.make_async_copy(zero_v, acc_sh.at[pl.ds(row0, _ZR)], zs).wait()
    plsc.subcore_barrier()

    def _chunk(i, b, q, load):
        _scat_wait(b)
        if load:
            _idx_load(i + 2, (q + 2) % 4)
        _idx_wait(q)
        _scat(b, q)

    _idx_wait(0)
    _scat(0, 0)
    _idx_wait(1)
    _scat(1, 1)

    def _quad(g, carry):
        i0 = 4 * g + 2
        _chunk(i0, 0, 2, True)
        _chunk(i0 + 1, 1, 3, True)
        _chunk(i0 + 2, 0, 0, True)
        _chunk(i0 + 3, 1, 1, True)
        return carry
    lax.fori_loop(0, (_NCH - 5) // 4, _quad, 0)

    _chunk(_NCH - 3, 0, 2, True)
    _chunk(_NCH - 2, 1, 3, False)
    _chunk(_NCH - 1, 0, 0, False)

    _scat_wait(1)
    _scat_wait(0)

    plsc.subcore_barrier()
    pltpu.sync_copy(acc_sh.at[pl.ds(row0, _RPT)],
                    out_hbm.at[c, pl.ds(row0, _RPT)])


def _tc_prep(degp, feats):
    def body(degp_ref, f_ref, inv_ref, g_ref):
        deg = degp_ref[0, :, 0:1] + degp_ref[1, :, 0:1]
        inv = lax.rsqrt(jnp.maximum(deg, 1.0))
        inv_ref[...] = jnp.broadcast_to(inv, (_BR, _D))
        g_ref[...] = f_ref[...] * inv

    return pl.pallas_call(
        body,
        grid=(_G,),
        in_specs=[
            pl.BlockSpec((_NC, _BR, _D), lambda i: (0, i, 0)),
            pl.BlockSpec((_BR, _D), lambda i: (i, 0)),
        ],
        out_specs=[
            pl.BlockSpec((_BR, _D), lambda i: (i, 0)),
            pl.BlockSpec((_BR, _D), lambda i: (i, 0)),
        ],
        out_shape=[
            jax.ShapeDtypeStruct((_N, _D), jnp.float32),
            jax.ShapeDtypeStruct((_N, _D), jnp.float32),
        ],
    )(degp, feats)


def _tc_layer1(h, s_part, inv, w01):
    def body(h_ref, s_ref, inv_ref, w_ref, out_ref, g2_ref):
        ssum = s_ref[0] + s_ref[1]
        inv_v = inv_ref[...]
        x1 = -(inv_v * ssum)
        out_ref[...] = (
            jnp.dot(h_ref[...], w_ref[0], preferred_element_type=jnp.float32)
            + jnp.dot(x1, w_ref[1], preferred_element_type=jnp.float32))
        g2_ref[...] = inv_v * x1

    return pl.pallas_call(
        body,
        grid=(_G,),
        in_specs=[
            pl.BlockSpec((_BR, _D), lambda i: (i, 0)),
            pl.BlockSpec((_NC, _BR, _D), lambda i: (0, i, 0)),
            pl.BlockSpec((_BR, _D), lambda i: (i, 0)),
            pl.BlockSpec((2, _D, _D), lambda i: (0, 0, 0)),
        ],
        out_specs=[
            pl.BlockSpec((_BR, _D), lambda i: (i, 0)),
            pl.BlockSpec((_BR, _D), lambda i: (i, 0)),
        ],
        out_shape=[
            jax.ShapeDtypeStruct((_N, _D), jnp.float32),
            jax.ShapeDtypeStruct((_N, _D), jnp.float32),
        ],
    )(h, s_part, inv, w01)


def _tc_layer2(out_acc, h, s_part, inv, w2, b):
    def body(o_ref, h_ref, s_ref, inv_ref, w_ref, b_ref, hn_ref, gn_ref):
        ssum = s_ref[0] + s_ref[1]
        inv_v = inv_ref[...]
        x2 = -2.0 * inv_v * ssum - h_ref[...]
        hn = jnp.maximum(
            o_ref[...]
            + jnp.dot(x2, w_ref[...], preferred_element_type=jnp.float32)
            + b_ref[...], 0.0)
        hn_ref[...] = hn
        gn_ref[...] = inv_v * hn

    return pl.pallas_call(
        body,
        grid=(_G,),
        in_specs=[
            pl.BlockSpec((_BR, _D), lambda i: (i, 0)),
            pl.BlockSpec((_BR, _D), lambda i: (i, 0)),
            pl.BlockSpec((_NC, _BR, _D), lambda i: (0, i, 0)),
            pl.BlockSpec((_BR, _D), lambda i: (i, 0)),
            pl.BlockSpec((_D, _D), lambda i: (0, 0)),
            pl.BlockSpec((1, _D), lambda i: (0, 0)),
        ],
        out_specs=[
            pl.BlockSpec((_BR, _D), lambda i: (i, 0)),
            pl.BlockSpec((_BR, _D), lambda i: (i, 0)),
        ],
        out_shape=[
            jax.ShapeDtypeStruct((_N, _D), jnp.float32),
            jax.ShapeDtypeStruct((_N, _D), jnp.float32),
        ],
    )(out_acc, h, s_part, inv, w2, b)


def _tc_head(h, w0, b0, w1, b1, w2, b2):
    def body(h_ref, w0_ref, b0_ref, w1_ref, b1_ref, w2_ref, b2_ref,
             out_ref, acc_ref):
        i = pl.program_id(0)

        @pl.when(i == 0)
        def _():
            acc_ref[...] = jnp.zeros_like(acc_ref)

        acc_ref[...] += jnp.sum(h_ref[...], axis=0, keepdims=True)

        @pl.when(i == _G - 1)
        def _():
            hg = acc_ref[...] * (1.0 / _N)
            x = jnp.maximum(
                jnp.dot(hg, w0_ref[...], preferred_element_type=jnp.float32)
                + b0_ref[...], 0.0)
            x = jnp.maximum(
                jnp.dot(x, w1_ref[...], preferred_element_type=jnp.float32)
                + b1_ref[...], 0.0)
            out_ref[...] = (
                jnp.dot(x, w2_ref[...], preferred_element_type=jnp.float32)
                + b2_ref[...])

    return pl.pallas_call(
        body,
        grid=(_G,),
        in_specs=[
            pl.BlockSpec((_BR, _D), lambda i: (i, 0)),
            pl.BlockSpec((_D, _D // 2), lambda i: (0, 0)),
            pl.BlockSpec((1, _D // 2), lambda i: (0, 0)),
            pl.BlockSpec((_D // 2, _D // 4), lambda i: (0, 0)),
            pl.BlockSpec((1, _D // 4), lambda i: (0, 0)),
            pl.BlockSpec((_D // 4, 40), lambda i: (0, 0)),
            pl.BlockSpec((1, 40), lambda i: (0, 0)),
        ],
        out_specs=pl.BlockSpec((1, 40), lambda i: (0, 0)),
        out_shape=jax.ShapeDtypeStruct((1, 40), jnp.float32),
        scratch_shapes=[pltpu.VMEM((1, _D), jnp.float32)],
    )(h, w0, b0, w1, b1, w2, b2)


def kernel(features, edge_index, cheb_W, cheb_b,
           mlp_W0, mlp_b0, mlp_W1, mlp_b1, mlp_W2, mlp_b2):
    src = edge_index[0]
    dst = edge_index[1]

    degp = _sc_degree(dst)
    inv, g = _tc_prep(degp, features)

    h = features
    n_layers = cheb_W.shape[0]
    for layer in range(n_layers):
        s1 = _sc_segsum(g, src, dst)
        out_acc, g2 = _tc_layer1(h, s1, inv, cheb_W[layer, 0:2])
        s2 = _sc_segsum(g2, src, dst)
        h, g = _tc_layer2(out_acc, h, s2, inv, cheb_W[layer, 2],
                          cheb_b[layer].reshape(1, -1))

    return _tc_head(h, mlp_W0, mlp_b0.reshape(1, -1),
                    mlp_W1, mlp_b1.reshape(1, -1),
                    mlp_W2, mlp_b2.reshape(1, -1))


# speedup vs baseline: 14.0870x; 1.0035x over previous
import functools

import jax
import jax.numpy as jnp
from jax import lax
from jax.experimental import pallas as pl
from jax.experimental.pallas import tpu as pltpu
from jax.experimental.pallas import tpu_sc as plsc

_N = 10000
_E = 320000
_D = 128
_NC = 2
_NS = 16
_NW = _NC * _NS
_EPT = _E // _NW
_C = 80
_NCH = _EPT // _C
_NP = 10240
_RPT = _NP // _NS
_ZR = 32
_NZ = _RPT // _ZR

_BR = 400
_G = _N // _BR


def _sc_mesh():
    return plsc.VectorSubcoreMesh(core_axis_name="c", subcore_axis_name="s")


@functools.partial(
    pl.kernel,
    out_type=jax.ShapeDtypeStruct((_NC, _NP, _D), jnp.float32),
    mesh=_sc_mesh(),
    scratch_types=[
        pltpu.VMEM((_EPT,), jnp.int32),
        pltpu.VMEM((_EPT,), jnp.int32),
        pltpu.VMEM((_C, _D), jnp.float32),
        pltpu.VMEM((_C, _D), jnp.float32),
        pltpu.VMEM((_ZR, _D), jnp.float32),
        pltpu.VMEM_SHARED((_NP, _D), jnp.float32),
        pltpu.SemaphoreType.DMA,
        pltpu.SemaphoreType.DMA,
        pltpu.SemaphoreType.DMA,
        pltpu.SemaphoreType.DMA,
        pltpu.SemaphoreType.DMA,
        pltpu.SemaphoreType.DMA,
    ],
)
def _sc_segsum(g_hbm, src_hbm, dst_hbm, out_hbm,
               sidx, didx, buf0, buf1, zero_v, acc_sh,
               isem, gs0, gs1, as0, as1, zs):
    c = lax.axis_index("c")
    s = lax.axis_index("s")
    base_e = (c * _NS + s) * _EPT

    bufs = (buf0, buf1)
    gsems = (gs0, gs1)
    asems = (as0, as1)

    pltpu.async_copy(src_hbm.at[pl.ds(base_e, _EPT)], sidx, isem)
    pltpu.async_copy(dst_hbm.at[pl.ds(base_e, _EPT)], didx, isem)

    def _gather(i, b):
        pltpu.async_copy(g_hbm.at[sidx.at[pl.ds(i * _C, _C)]], bufs[b],
                         gsems[b])

    def _gather_wait(b):
        pltpu.make_async_copy(g_hbm.at[pl.ds(0, _C)], bufs[b],
                              gsems[b]).wait()

    def _scat(i, b):
        pltpu.async_copy(bufs[b], acc_sh.at[didx.at[pl.ds(i * _C, _C)]],
                         asems[b], add=True)

    def _scat_wait(b):
        pltpu.make_async_copy(bufs[b], acc_sh.at[pl.ds(0, _C)],
                              asems[b]).wait()

    def _zrow(i, carry):
        for j in range(_D // 16):
            zero_v[i, pl.ds(j * 16, 16)] = jnp.zeros((16,), jnp.float32)
        return carry
    lax.fori_loop(0, _ZR, _zrow, 0)

    row0 = s * _RPT
    for t in range(_NZ):
        pltpu.async_copy(zero_v, acc_sh.at[pl.ds(row0 + t * _ZR, _ZR)], zs)

    pltpu.make_async_copy(src_hbm.at[pl.ds(0, _EPT)], sidx, isem).wait()
    pltpu.make_async_copy(dst_hbm.at[pl.ds(0, _EPT)], didx, isem).wait()
    _gather(0, 0)

    for t in range(_NZ):
        pltpu.make_async_copy(zero_v, acc_sh.at[pl.ds(row0, _ZR)], zs).wait()
    plsc.subcore_barrier()

    def _chunk(i, b):
        _scat_wait(b)
        _gather(i, b)
        _gather_wait(1 - b)
        _scat(i - 1, 1 - b)

    _gather(1, 1)
    _gather_wait(0)
    _scat(0, 0)
    _chunk(2, 0)

    def _pair(g, carry):
        i0 = 2 * g + 3
        _chunk(i0, 1)
        _chunk(i0 + 1, 0)
        return carry
    lax.fori_loop(0, (_NCH - 3) // 2, _pair, 0)

    _gather_wait(0)
    _scat(_NCH - 1, 0)
    _scat_wait(1)
    _scat_wait(0)

    plsc.subcore_barrier()
    pltpu.sync_copy(acc_sh.at[pl.ds(row0, _RPT)],
                    out_hbm.at[c, pl.ds(row0, _RPT)])


@functools.partial(
    pl.kernel,
    out_type=jax.ShapeDtypeStruct((_NC, _NP, _D), jnp.float32),
    mesh=_sc_mesh(),
    scratch_types=[
        pltpu.VMEM((_EPT,), jnp.int32),
        pltpu.VMEM((_C, _D), jnp.float32),
        pltpu.VMEM((_ZR, _D), jnp.float32),
        pltpu.VMEM_SHARED((_NP, _D), jnp.float32),
        pltpu.SemaphoreType.DMA,
        pltpu.SemaphoreType.DMA,
        pltpu.SemaphoreType.DMA,
        pltpu.SemaphoreType.DMA,
    ],
)
def _sc_degree(dst_hbm, out_hbm,
               didx, ones_v, zero_v, acc_sh,
               isem, as0, as1, zs):
    c = lax.axis_index("c")
    s = lax.axis_index("s")
    base_e = (c * _NS + s) * _EPT

    asems = (as0, as1)

    pltpu.async_copy(dst_hbm.at[pl.ds(base_e, _EPT)], didx, isem)

    def _scat(i, b):
        pltpu.async_copy(ones_v, acc_sh.at[didx.at[pl.ds(i * _C, _C)]],
                         asems[b], add=True)

    def _scat_wait(b):
        pltpu.make_async_copy(ones_v, acc_sh.at[pl.ds(0, _C)],
                              asems[b]).wait()

    def _zrow(i, carry):
        for j in range(_D // 16):
            zero_v[i, pl.ds(j * 16, 16)] = jnp.zeros((16,), jnp.float32)
        return carry
    lax.fori_loop(0, _ZR, _zrow, 0)

    def _orow(i, carry):
        for j in range(_D // 16):
            ones_v[i, pl.ds(j * 16, 16)] = jnp.ones((16,), jnp.float32)
        return carry
    lax.fori_loop(0, _C, _orow, 0)

    row0 = s * _RPT
    for t in range(_NZ):
        pltpu.async_copy(zero_v, acc_sh.at[pl.ds(row0 + t * _ZR, _ZR)], zs)
    for t in range(_NZ):
        pltpu.make_async_copy(zero_v, acc_sh.at[pl.ds(row0, _ZR)], zs).wait()
    pltpu.make_async_copy(dst_hbm.at[pl.ds(0, _EPT)], didx, isem).wait()
    plsc.subcore_barrier()

    _scat(0, 0)
    _scat(1, 1)

    def _pair(g, carry):
        i0 = 2 * g + 2
        _scat_wait(0)
        _scat(i0, 0)
        _scat_wait(1)
        _scat(i0 + 1, 1)
        return carry
    lax.fori_loop(0, (_NCH - 3) // 2, _pair, 0)

    _scat_wait(0)
    _scat(_NCH - 1, 0)
    _scat_wait(1)
    _scat_wait(0)

    plsc.subcore_barrier()
    pltpu.sync_copy(acc_sh.at[pl.ds(row0, _RPT)],
                    out_hbm.at[c, pl.ds(row0, _RPT)])


def _tc_prep(degp, feats):
    def body(degp_ref, f_ref, inv_ref, g_ref):
        deg = degp_ref[0, :, 0:1] + degp_ref[1, :, 0:1]
        inv = lax.rsqrt(jnp.maximum(deg, 1.0))
        inv_ref[...] = jnp.broadcast_to(inv, (_BR, _D))
        g_ref[...] = f_ref[...] * inv

    return pl.pallas_call(
        body,
        grid=(_G,),
        in_specs=[
            pl.BlockSpec((_NC, _BR, _D), lambda i: (0, i, 0)),
            pl.BlockSpec((_BR, _D), lambda i: (i, 0)),
        ],
        out_specs=[
            pl.BlockSpec((_BR, _D), lambda i: (i, 0)),
            pl.BlockSpec((_BR, _D), lambda i: (i, 0)),
        ],
        out_shape=[
            jax.ShapeDtypeStruct((_N, _D), jnp.float32),
            jax.ShapeDtypeStruct((_N, _D), jnp.float32),
        ],
    )(degp, feats)


def _tc_layer1(h, s_part, inv, w01):
    def body(h_ref, s_ref, inv_ref, w_ref, out_ref, g2_ref):
        ssum = s_ref[0] + s_ref[1]
        inv_v = inv_ref[...]
        x1 = -(inv_v * ssum)
        out_ref[...] = (
            jnp.dot(h_ref[...], w_ref[0], preferred_element_type=jnp.float32)
            + jnp.dot(x1, w_ref[1], preferred_element_type=jnp.float32))
        g2_ref[...] = inv_v * x1

    return pl.pallas_call(
        body,
        grid=(_G,),
        in_specs=[
            pl.BlockSpec((_BR, _D), lambda i: (i, 0)),
            pl.BlockSpec((_NC, _BR, _D), lambda i: (0, i, 0)),
            pl.BlockSpec((_BR, _D), lambda i: (i, 0)),
            pl.BlockSpec((2, _D, _D), lambda i: (0, 0, 0)),
        ],
        out_specs=[
            pl.BlockSpec((_BR, _D), lambda i: (i, 0)),
            pl.BlockSpec((_BR, _D), lambda i: (i, 0)),
        ],
        out_shape=[
            jax.ShapeDtypeStruct((_N, _D), jnp.float32),
            jax.ShapeDtypeStruct((_N, _D), jnp.float32),
        ],
    )(h, s_part, inv, w01)


def _tc_layer2(out_acc, h, s_part, inv, w2, b):
    def body(o_ref, h_ref, s_ref, inv_ref, w_ref, b_ref, hn_ref, gn_ref):
        ssum = s_ref[0] + s_ref[1]
        inv_v = inv_ref[...]
        x2 = -2.0 * inv_v * ssum - h_ref[...]
        hn = jnp.maximum(
            o_ref[...]
            + jnp.dot(x2, w_ref[...], preferred_element_type=jnp.float32)
            + b_ref[...], 0.0)
        hn_ref[...] = hn
        gn_ref[...] = inv_v * hn

    return pl.pallas_call(
        body,
        grid=(_G,),
        in_specs=[
            pl.BlockSpec((_BR, _D), lambda i: (i, 0)),
            pl.BlockSpec((_BR, _D), lambda i: (i, 0)),
            pl.BlockSpec((_NC, _BR, _D), lambda i: (0, i, 0)),
            pl.BlockSpec((_BR, _D), lambda i: (i, 0)),
            pl.BlockSpec((_D, _D), lambda i: (0, 0)),
            pl.BlockSpec((1, _D), lambda i: (0, 0)),
        ],
        out_specs=[
            pl.BlockSpec((_BR, _D), lambda i: (i, 0)),
            pl.BlockSpec((_BR, _D), lambda i: (i, 0)),
        ],
        out_shape=[
            jax.ShapeDtypeStruct((_N, _D), jnp.float32),
            jax.ShapeDtypeStruct((_N, _D), jnp.float32),
        ],
    )(out_acc, h, s_part, inv, w2, b)


def _tc_head(h, w0, b0, w1, b1, w2, b2):
    def body(h_ref, w0_ref, b0_ref, w1_ref, b1_ref, w2_ref, b2_ref,
             out_ref, acc_ref):
        i = pl.program_id(0)

        @pl.when(i == 0)
        def _():
            acc_ref[...] = jnp.zeros_like(acc_ref)

        acc_ref[...] += jnp.sum(h_ref[...], axis=0, keepdims=True)

        @pl.when(i == _G - 1)
        def _():
            hg = acc_ref[...] * (1.0 / _N)
            x = jnp.maximum(
                jnp.dot(hg, w0_ref[...], preferred_element_type=jnp.float32)
                + b0_ref[...], 0.0)
            x = jnp.maximum(
                jnp.dot(x, w1_ref[...], preferred_element_type=jnp.float32)
                + b1_ref[...], 0.0)
            out_ref[...] = (
                jnp.dot(x, w2_ref[...], preferred_element_type=jnp.float32)
                + b2_ref[...])

    return pl.pallas_call(
        body,
        grid=(_G,),
        in_specs=[
            pl.BlockSpec((_BR, _D), lambda i: (i, 0)),
            pl.BlockSpec((_D, _D // 2), lambda i: (0, 0)),
            pl.BlockSpec((1, _D // 2), lambda i: (0, 0)),
            pl.BlockSpec((_D // 2, _D // 4), lambda i: (0, 0)),
            pl.BlockSpec((1, _D // 4), lambda i: (0, 0)),
            pl.BlockSpec((_D // 4, 40), lambda i: (0, 0)),
            pl.BlockSpec((1, 40), lambda i: (0, 0)),
        ],
        out_specs=pl.BlockSpec((1, 40), lambda i: (0, 0)),
        out_shape=jax.ShapeDtypeStruct((1, 40), jnp.float32),
        scratch_shapes=[pltpu.VMEM((1, _D), jnp.float32)],
    )(h, w0, b0, w1, b1, w2, b2)


def kernel(features, edge_index, cheb_W, cheb_b,
           mlp_W0, mlp_b0, mlp_W1, mlp_b1, mlp_W2, mlp_b2):
    src = edge_index[0]
    dst = edge_index[1]

    degp = _sc_degree(dst)
    inv, g = _tc_prep(degp, features)

    h = features
    n_layers = cheb_W.shape[0]
    for layer in range(n_layers):
        s1 = _sc_segsum(g, src, dst)
        out_acc, g2 = _tc_layer1(h, s1, inv, cheb_W[layer, 0:2])
        s2 = _sc_segsum(g2, src, dst)
        h, g = _tc_layer2(out_acc, h, s2, inv, cheb_W[layer, 2],
                          cheb_b[layer].reshape(1, -1))

    return _tc_head(h, mlp_W0, mlp_b0.reshape(1, -1),
                    mlp_W1, mlp_b1.reshape(1, -1),
                    mlp_W2, mlp_b2.reshape(1, -1))


# speedup vs baseline: 14.0961x; 1.0006x over previous
import functools

import jax
import jax.numpy as jnp
from jax import lax
from jax.experimental import pallas as pl
from jax.experimental.pallas import tpu as pltpu
from jax.experimental.pallas import tpu_sc as plsc

_N = 10000
_E = 320000
_D = 128
_NC = 2
_NS = 16
_NW = _NC * _NS
_EPT = _E // _NW
_C = 80
_NCH = _EPT // _C
_NP = 10240
_RPT = _NP // _NS
_ZR = 32
_NZ = _RPT // _ZR

_BR = 400
_G = _N // _BR


def _sc_mesh():
    return plsc.VectorSubcoreMesh(core_axis_name="c", subcore_axis_name="s")


@functools.partial(
    pl.kernel,
    out_type=jax.ShapeDtypeStruct((_NC, _NP, _D), jnp.float32),
    mesh=_sc_mesh(),
    scratch_types=[
        pltpu.VMEM((_EPT,), jnp.int32),
        pltpu.VMEM((_EPT,), jnp.int32),
        pltpu.VMEM((_C, _D), jnp.float32),
        pltpu.VMEM((_C, _D), jnp.float32),
        pltpu.VMEM((_ZR, _D), jnp.float32),
        pltpu.VMEM_SHARED((_NP, _D), jnp.float32),
        pltpu.SemaphoreType.DMA,
        pltpu.SemaphoreType.DMA,
        pltpu.SemaphoreType.DMA,
        pltpu.SemaphoreType.DMA,
        pltpu.SemaphoreType.DMA,
        pltpu.SemaphoreType.DMA,
    ],
)
def _sc_segsum(g_hbm, src_hbm, dst_hbm, out_hbm,
               sidx, didx, buf0, buf1, zero_v, acc_sh,
               isem, gs0, gs1, as0, as1, zs):
    c = lax.axis_index("c")
    s = lax.axis_index("s")
    base_e = (c * _NS + s) * _EPT

    bufs = (buf0, buf1)
    gsems = (gs0, gs1)
    asems = (as0, as1)

    pltpu.async_copy(src_hbm.at[pl.ds(base_e, _EPT)], sidx, isem)
    pltpu.async_copy(dst_hbm.at[pl.ds(base_e, _EPT)], didx, isem)

    def _gather(i, b):
        pltpu.async_copy(g_hbm.at[sidx.at[pl.ds(i * _C, _C)]], bufs[b],
                         gsems[b])

    def _gather_wait(b):
        pltpu.make_async_copy(g_hbm.at[pl.ds(0, _C)], bufs[b],
                              gsems[b]).wait()

    def _scat(i, b):
        pltpu.async_copy(bufs[b], acc_sh.at[didx.at[pl.ds(i * _C, _C)]],
                         asems[b], add=True)

    def _scat_wait(b):
        pltpu.make_async_copy(bufs[b], acc_sh.at[pl.ds(0, _C)],
                              asems[b]).wait()

    def _zrow(i, carry):
        for j in range(_D // 16):
            zero_v[i, pl.ds(j * 16, 16)] = jnp.zeros((16,), jnp.float32)
        return carry
    lax.fori_loop(0, _ZR, _zrow, 0)

    row0 = s * _RPT
    for t in range(_NZ):
        pltpu.async_copy(zero_v, acc_sh.at[pl.ds(row0 + t * _ZR, _ZR)], zs)

    pltpu.make_async_copy(src_hbm.at[pl.ds(0, _EPT)], sidx, isem).wait()
    pltpu.make_async_copy(dst_hbm.at[pl.ds(0, _EPT)], didx, isem).wait()
    _gather(0, 0)

    for t in range(_NZ):
        pltpu.make_async_copy(zero_v, acc_sh.at[pl.ds(row0, _ZR)], zs).wait()
    plsc.subcore_barrier()

    def _chunk(i, b):
        _scat_wait(b)
        _gather(i, b)
        _gather_wait(1 - b)
        _scat(i - 1, 1 - b)

    _gather(1, 1)
    _gather_wait(0)
    _scat(0, 0)
    _chunk(2, 0)

    def _pair(g, carry):
        i0 = 2 * g + 3
        _chunk(i0, 1)
        _chunk(i0 + 1, 0)
        return carry
    lax.fori_loop(0, (_NCH - 3) // 2, _pair, 0)

    if (_NCH - 3) % 2:
        _chunk(_NCH - 1, 1)

    lastb = (_NCH - 1) % 2
    _gather_wait(lastb)
    _scat(_NCH - 1, lastb)
    _scat_wait(1 - lastb)
    _scat_wait(lastb)

    plsc.subcore_barrier()
    pltpu.sync_copy(acc_sh.at[pl.ds(row0, _RPT)],
                    out_hbm.at[c, pl.ds(row0, _RPT)])


@functools.partial(
    pl.kernel,
    out_type=jax.ShapeDtypeStruct((_NC, _NP, _D), jnp.float32),
    mesh=_sc_mesh(),
    scratch_types=[
        pltpu.VMEM((_EPT,), jnp.int32),
        pltpu.VMEM((_C, _D), jnp.float32),
        pltpu.VMEM((_ZR, _D), jnp.float32),
        pltpu.VMEM_SHARED((_NP, _D), jnp.float32),
        pltpu.SemaphoreType.DMA,
        pltpu.SemaphoreType.DMA,
        pltpu.SemaphoreType.DMA,
        pltpu.SemaphoreType.DMA,
    ],
)
def _sc_degree(dst_hbm, out_hbm,
               didx, ones_v, zero_v, acc_sh,
               isem, as0, as1, zs):
    c = lax.axis_index("c")
    s = lax.axis_index("s")
    base_e = (c * _NS + s) * _EPT

    asems = (as0, as1)

    pltpu.async_copy(dst_hbm.at[pl.ds(base_e, _EPT)], didx, isem)

    def _scat(i, b):
        pltpu.async_copy(ones_v, acc_sh.at[didx.at[pl.ds(i * _C, _C)]],
                         asems[b], add=True)

    def _scat_wait(b):
        pltpu.make_async_copy(ones_v, acc_sh.at[pl.ds(0, _C)],
                              asems[b]).wait()

    def _zrow(i, carry):
        for j in range(_D // 16):
            zero_v[i, pl.ds(j * 16, 16)] = jnp.zeros((16,), jnp.float32)
        return carry
    lax.fori_loop(0, _ZR, _zrow, 0)

    def _orow(i, carry):
        for j in range(_D // 16):
            ones_v[i, pl.ds(j * 16, 16)] = jnp.ones((16,), jnp.float32)
        return carry
    lax.fori_loop(0, _C, _orow, 0)

    row0 = s * _RPT
    for t in range(_NZ):
        pltpu.async_copy(zero_v, acc_sh.at[pl.ds(row0 + t * _ZR, _ZR)], zs)
    for t in range(_NZ):
        pltpu.make_async_copy(zero_v, acc_sh.at[pl.ds(row0, _ZR)], zs).wait()
    pltpu.make_async_copy(dst_hbm.at[pl.ds(0, _EPT)], didx, isem).wait()
    plsc.subcore_barrier()

    _scat(0, 0)
    _scat(1, 1)

    def _pair(g, carry):
        i0 = 2 * g + 2
        _scat_wait(0)
        _scat(i0, 0)
        _scat_wait(1)
        _scat(i0 + 1, 1)
        return carry
    lax.fori_loop(0, (_NCH - 2) // 2, _pair, 0)

    if (_NCH - 2) % 2:
        _scat_wait(0)
        _scat(_NCH - 1, 0)

    lastb = (_NCH - 1) % 2
    _scat_wait(1 - lastb)
    _scat_wait(lastb)

    plsc.subcore_barrier()
    pltpu.sync_copy(acc_sh.at[pl.ds(row0, _RPT)],
                    out_hbm.at[c, pl.ds(row0, _RPT)])


def _tc_prep(degp, feats):
    def body(degp_ref, f_ref, inv_ref, g_ref):
        deg = degp_ref[0, :, 0:1] + degp_ref[1, :, 0:1]
        inv = lax.rsqrt(jnp.maximum(deg, 1.0))
        inv_ref[...] = jnp.broadcast_to(inv, (_BR, _D))
        g_ref[...] = f_ref[...] * inv

    return pl.pallas_call(
        body,
        grid=(_G,),
        in_specs=[
            pl.BlockSpec((_NC, _BR, _D), lambda i: (0, i, 0)),
            pl.BlockSpec((_BR, _D), lambda i: (i, 0)),
        ],
        out_specs=[
            pl.BlockSpec((_BR, _D), lambda i: (i, 0)),
            pl.BlockSpec((_BR, _D), lambda i: (i, 0)),
        ],
        out_shape=[
            jax.ShapeDtypeStruct((_N, _D), jnp.float32),
            jax.ShapeDtypeStruct((_N, _D), jnp.float32),
        ],
    )(degp, feats)


def _tc_layer1(h, s_part, inv, w01):
    def body(h_ref, s_ref, inv_ref, w_ref, out_ref, g2_ref):
        ssum = s_ref[0] + s_ref[1]
        inv_v = inv_ref[...]
        x1 = -(inv_v * ssum)
        out_ref[...] = (
            jnp.dot(h_ref[...], w_ref[0], preferred_element_type=jnp.float32)
            + jnp.dot(x1, w_ref[1], preferred_element_type=jnp.float32))
        g2_ref[...] = inv_v * x1

    return pl.pallas_call(
        body,
        grid=(_G,),
        in_specs=[
            pl.BlockSpec((_BR, _D), lambda i: (i, 0)),
            pl.BlockSpec((_NC, _BR, _D), lambda i: (0, i, 0)),
            pl.BlockSpec((_BR, _D), lambda i: (i, 0)),
            pl.BlockSpec((2, _D, _D), lambda i: (0, 0, 0)),
        ],
        out_specs=[
            pl.BlockSpec((_BR, _D), lambda i: (i, 0)),
            pl.BlockSpec((_BR, _D), lambda i: (i, 0)),
        ],
        out_shape=[
            jax.ShapeDtypeStruct((_N, _D), jnp.float32),
            jax.ShapeDtypeStruct((_N, _D), jnp.float32),
        ],
    )(h, s_part, inv, w01)


def _tc_layer2(out_acc, h, s_part, inv, w2, b):
    def body(o_ref, h_ref, s_ref, inv_ref, w_ref, b_ref, hn_ref, gn_ref):
        ssum = s_ref[0] + s_ref[1]
        inv_v = inv_ref[...]
        x2 = -2.0 * inv_v * ssum - h_ref[...]
        hn = jnp.maximum(
            o_ref[...]
            + jnp.dot(x2, w_ref[...], preferred_element_type=jnp.float32)
            + b_ref[...], 0.0)
        hn_ref[...] = hn
        gn_ref[...] = inv_v * hn

    return pl.pallas_call(
        body,
        grid=(_G,),
        in_specs=[
            pl.BlockSpec((_BR, _D), lambda i: (i, 0)),
            pl.BlockSpec((_BR, _D), lambda i: (i, 0)),
            pl.BlockSpec((_NC, _BR, _D), lambda i: (0, i, 0)),
            pl.BlockSpec((_BR, _D), lambda i: (i, 0)),
            pl.BlockSpec((_D, _D), lambda i: (0, 0)),
            pl.BlockSpec((1, _D), lambda i: (0, 0)),
        ],
        out_specs=[
            pl.BlockSpec((_BR, _D), lambda i: (i, 0)),
            pl.BlockSpec((_BR, _D), lambda i: (i, 0)),
        ],
        out_shape=[
            jax.ShapeDtypeStruct((_N, _D), jnp.float32),
            jax.ShapeDtypeStruct((_N, _D), jnp.float32),
        ],
    )(out_acc, h, s_part, inv, w2, b)


def _tc_head(h, w0, b0, w1, b1, w2, b2):
    def body(h_ref, w0_ref, b0_ref, w1_ref, b1_ref, w2_ref, b2_ref,
             out_ref, acc_ref):
        i = pl.program_id(0)

        @pl.when(i == 0)
        def _():
            acc_ref[...] = jnp.zeros_like(acc_ref)

        acc_ref[...] += jnp.sum(h_ref[...], axis=0, keepdims=True)

        @pl.when(i == _G - 1)
        def _():
            hg = acc_ref[...] * (1.0 / _N)
            x = jnp.maximum(
                jnp.dot(hg, w0_ref[...], preferred_element_type=jnp.float32)
                + b0_ref[...], 0.0)
            x = jnp.maximum(
                jnp.dot(x, w1_ref[...], preferred_element_type=jnp.float32)
                + b1_ref[...], 0.0)
            out_ref[...] = (
                jnp.dot(x, w2_ref[...], preferred_element_type=jnp.float32)
                + b2_ref[...])

    return pl.pallas_call(
        body,
        grid=(_G,),
        in_specs=[
            pl.BlockSpec((_BR, _D), lambda i: (i, 0)),
            pl.BlockSpec((_D, _D // 2), lambda i: (0, 0)),
            pl.BlockSpec((1, _D // 2), lambda i: (0, 0)),
            pl.BlockSpec((_D // 2, _D // 4), lambda i: (0, 0)),
            pl.BlockSpec((1, _D // 4), lambda i: (0, 0)),
            pl.BlockSpec((_D // 4, 40), lambda i: (0, 0)),
            pl.BlockSpec((1, 40), lambda i: (0, 0)),
        ],
        out_specs=pl.BlockSpec((1, 40), lambda i: (0, 0)),
        out_shape=jax.ShapeDtypeStruct((1, 40), jnp.float32),
        scratch_shapes=[pltpu.VMEM((1, _D), jnp.float32)],
    )(h, w0, b0, w1, b1, w2, b2)


def kernel(features, edge_index, cheb_W, cheb_b,
           mlp_W0, mlp_b0, mlp_W1, mlp_b1, mlp_W2, mlp_b2):
    src = edge_index[0]
    dst = edge_index[1]

    degp = _sc_degree(dst)
    inv, g = _tc_prep(degp, features)

    h = features
    n_layers = cheb_W.shape[0]
    for layer in range(n_layers):
        s1 = _sc_segsum(g, src, dst)
        out_acc, g2 = _tc_layer1(h, s1, inv, cheb_W[layer, 0:2])
        s2 = _sc_segsum(g2, src, dst)
        h, g = _tc_layer2(out_acc, h, s2, inv, cheb_W[layer, 2],
                          cheb_b[layer].reshape(1, -1))

    return _tc_head(h, mlp_W0, mlp_b0.reshape(1, -1),
                    mlp_W1, mlp_b1.reshape(1, -1),
                    mlp_W2, mlp_b2.reshape(1, -1))


# speedup vs baseline: 15.0706x; 1.0691x over previous
import functools

import jax
import jax.numpy as jnp
from jax import lax
from jax.experimental import pallas as pl
from jax.experimental.pallas import tpu as pltpu
from jax.experimental.pallas import tpu_sc as plsc

_N = 10000
_E = 320000
_D = 128
_NC = 2
_NS = 16
_NW = _NC * _NS
_EPT = _E // _NW
_C = 40
_NCH = _EPT // _C
_NP = 10240
_RPT = _NP // _NS
_ZR = 8
_NZ = _RPT // _ZR

_BR = 400
_G = _N // _BR


def _sc_mesh():
    return plsc.VectorSubcoreMesh(core_axis_name="c", subcore_axis_name="s")


@functools.partial(
    pl.kernel,
    out_type=jax.ShapeDtypeStruct((_NC, _NP, _D), jnp.float32),
    mesh=_sc_mesh(),
    scratch_types=[
        pltpu.VMEM((_EPT,), jnp.int32),
        pltpu.VMEM((_EPT,), jnp.int32),
        pltpu.VMEM((_C, _D), jnp.float32),
        pltpu.VMEM((_C, _D), jnp.float32),
        pltpu.VMEM((_C, _D), jnp.float32),
        pltpu.VMEM((_C, _D), jnp.float32),
        pltpu.VMEM((_ZR, _D), jnp.float32),
        pltpu.VMEM_SHARED((_NP, _D), jnp.float32),
        pltpu.SemaphoreType.DMA,
        pltpu.SemaphoreType.DMA,
        pltpu.SemaphoreType.DMA,
        pltpu.SemaphoreType.DMA,
        pltpu.SemaphoreType.DMA,
        pltpu.SemaphoreType.DMA,
        pltpu.SemaphoreType.DMA,
        pltpu.SemaphoreType.DMA,
        pltpu.SemaphoreType.DMA,
        pltpu.SemaphoreType.DMA,
    ],
)
def _sc_segsum(g_hbm, src_hbm, dst_hbm, out_hbm,
               sidx, didx, buf0, buf1, buf2, buf3, zero_v, acc_sh,
               isem, gs0, gs1, gs2, gs3, as0, as1, as2, as3, zs):
    c = lax.axis_index("c")
    s = lax.axis_index("s")
    base_e = (c * _NS + s) * _EPT

    bufs = (buf0, buf1, buf2, buf3)
    gsems = (gs0, gs1, gs2, gs3)
    asems = (as0, as1, as2, as3)

    pltpu.async_copy(src_hbm.at[pl.ds(base_e, _EPT)], sidx, isem)
    pltpu.async_copy(dst_hbm.at[pl.ds(base_e, _EPT)], didx, isem)

    def _gather(i, b):
        pltpu.async_copy(g_hbm.at[sidx.at[pl.ds(i * _C, _C)]], bufs[b],
                         gsems[b])

    def _gather_wait(b):
        pltpu.make_async_copy(g_hbm.at[pl.ds(0, _C)], bufs[b],
                              gsems[b]).wait()

    def _scat(i, b):
        pltpu.async_copy(bufs[b], acc_sh.at[didx.at[pl.ds(i * _C, _C)]],
                         asems[b], add=True)

    def _scat_wait(b):
        pltpu.make_async_copy(bufs[b], acc_sh.at[pl.ds(0, _C)],
                              asems[b]).wait()

    def _zrow(i, carry):
        for j in range(_D // 16):
            zero_v[i, pl.ds(j * 16, 16)] = jnp.zeros((16,), jnp.float32)
        return carry
    lax.fori_loop(0, _ZR, _zrow, 0)

    row0 = s * _RPT
    for t in range(_NZ):
        pltpu.async_copy(zero_v, acc_sh.at[pl.ds(row0 + t * _ZR, _ZR)], zs)

    pltpu.make_async_copy(src_hbm.at[pl.ds(0, _EPT)], sidx, isem).wait()
    pltpu.make_async_copy(dst_hbm.at[pl.ds(0, _EPT)], didx, isem).wait()
    _gather(0, 0)
    _gather(1, 1)
    _gather(2, 2)

    for t in range(_NZ):
        pltpu.make_async_copy(zero_v, acc_sh.at[pl.ds(row0, _ZR)], zs).wait()
    plsc.subcore_barrier()

    def _body(i, b):
        _gather_wait(b)
        _scat(i, b)
        _scat_wait((b + 3) % 4)
        _gather(i + 3, (b + 3) % 4)

    _gather_wait(0)
    _scat(0, 0)
    _gather(3, 3)

    n_groups = (_NCH - 4) // 4

    def _group(g, carry):
        i0 = 4 * g + 1
        _body(i0, 1)
        _body(i0 + 1, 2)
        _body(i0 + 2, 3)
        _body(i0 + 3, 0)
        return carry
    lax.fori_loop(0, n_groups, _group, 0)

    for i in range(1 + 4 * n_groups, _NCH - 3):
        _body(i, i % 4)

    for i in range(_NCH - 3, _NCH):
        b = i % 4
        _gather_wait(b)
        _scat(i, b)
        _scat_wait((b + 3) % 4)

    _scat_wait((_NCH - 1) % 4)

    plsc.subcore_barrier()
    pltpu.sync_copy(acc_sh.at[pl.ds(row0, _RPT)],
                    out_hbm.at[c, pl.ds(row0, _RPT)])


@functools.partial(
    pl.kernel,
    out_type=jax.ShapeDtypeStruct((_NC, _NP, _D), jnp.float32),
    mesh=_sc_mesh(),
    scratch_types=[
        pltpu.VMEM((_EPT,), jnp.int32),
        pltpu.VMEM((_C, _D), jnp.float32),
        pltpu.VMEM((_ZR, _D), jnp.float32),
        pltpu.VMEM_SHARED((_NP, _D), jnp.float32),
        pltpu.SemaphoreType.DMA,
        pltpu.SemaphoreType.DMA,
        pltpu.SemaphoreType.DMA,
        pltpu.SemaphoreType.DMA,
    ],
)
def _sc_degree(dst_hbm, out_hbm,
               didx, ones_v, zero_v, acc_sh,
               isem, as0, as1, zs):
    c = lax.axis_index("c")
    s = lax.axis_index("s")
    base_e = (c * _NS + s) * _EPT

    asems = (as0, as1)

    pltpu.async_copy(dst_hbm.at[pl.ds(base_e, _EPT)], didx, isem)

    def _scat(i, b):
        pltpu.async_copy(ones_v, acc_sh.at[didx.at[pl.ds(i * _C, _C)]],
                         asems[b], add=True)

    def _scat_wait(b):
        pltpu.make_async_copy(ones_v, acc_sh.at[pl.ds(0, _C)],
                              asems[b]).wait()

    def _zrow(i, carry):
        for j in range(_D // 16):
            zero_v[i, pl.ds(j * 16, 16)] = jnp.zeros((16,), jnp.float32)
        return carry
    lax.fori_loop(0, _ZR, _zrow, 0)

    def _orow(i, carry):
        for j in range(_D // 16):
            ones_v[i, pl.ds(j * 16, 16)] = jnp.ones((16,), jnp.float32)
        return carry
    lax.fori_loop(0, _C, _orow, 0)

    row0 = s * _RPT
    for t in range(_NZ):
        pltpu.async_copy(zero_v, acc_sh.at[pl.ds(row0 + t * _ZR, _ZR)], zs)
    for t in range(_NZ):
        pltpu.make_async_copy(zero_v, acc_sh.at[pl.ds(row0, _ZR)], zs).wait()
    pltpu.make_async_copy(dst_hbm.at[pl.ds(0, _EPT)], didx, isem).wait()
    plsc.subcore_barrier()

    _scat(0, 0)
    _scat(1, 1)

    def _pair(g, carry):
        i0 = 2 * g + 2
        _scat_wait(0)
        _scat(i0, 0)
        _scat_wait(1)
        _scat(i0 + 1, 1)
        return carry
    lax.fori_loop(0, (_NCH - 2) // 2, _pair, 0)

    if (_NCH - 2) % 2:
        _scat_wait(0)
        _scat(_NCH - 1, 0)

    lastb = (_NCH - 1) % 2
    _scat_wait(1 - lastb)
    _scat_wait(lastb)

    plsc.subcore_barrier()
    pltpu.sync_copy(acc_sh.at[pl.ds(row0, _RPT)],
                    out_hbm.at[c, pl.ds(row0, _RPT)])


def _tc_prep(degp, feats):
    def body(degp_ref, f_ref, inv_ref, g_ref):
        deg = degp_ref[0, :, 0:1] + degp_ref[1, :, 0:1]
        inv = lax.rsqrt(jnp.maximum(deg, 1.0))
        inv_ref[...] = jnp.broadcast_to(inv, (_BR, _D))
        g_ref[...] = f_ref[...] * inv

    return pl.pallas_call(
        body,
        grid=(_G,),
        in_specs=[
            pl.BlockSpec((_NC, _BR, _D), lambda i: (0, i, 0)),
            pl.BlockSpec((_BR, _D), lambda i: (i, 0)),
        ],
        out_specs=[
            pl.BlockSpec((_BR, _D), lambda i: (i, 0)),
            pl.BlockSpec((_BR, _D), lambda i: (i, 0)),
        ],
        out_shape=[
            jax.ShapeDtypeStruct((_N, _D), jnp.float32),
            jax.ShapeDtypeStruct((_N, _D), jnp.float32),
        ],
    )(degp, feats)


def _tc_layer1(h, s_part, inv, w01):
    def body(h_ref, s_ref, inv_ref, w_ref, out_ref, g2_ref):
        ssum = s_ref[0] + s_ref[1]
        inv_v = inv_ref[...]
        x1 = -(inv_v * ssum)
        out_ref[...] = (
            jnp.dot(h_ref[...], w_ref[0], preferred_element_type=jnp.float32)
            + jnp.dot(x1, w_ref[1], preferred_element_type=jnp.float32))
        g2_ref[...] = inv_v * x1

    return pl.pallas_call(
        body,
        grid=(_G,),
        in_specs=[
            pl.BlockSpec((_BR, _D), lambda i: (i, 0)),
            pl.BlockSpec((_NC, _BR, _D), lambda i: (0, i, 0)),
            pl.BlockSpec((_BR, _D), lambda i: (i, 0)),
            pl.BlockSpec((2, _D, _D), lambda i: (0, 0, 0)),
        ],
        out_specs=[
            pl.BlockSpec((_BR, _D), lambda i: (i, 0)),
            pl.BlockSpec((_BR, _D), lambda i: (i, 0)),
        ],
        out_shape=[
            jax.ShapeDtypeStruct((_N, _D), jnp.float32),
            jax.ShapeDtypeStruct((_N, _D), jnp.float32),
        ],
    )(h, s_part, inv, w01)


def _tc_layer2(out_acc, h, s_part, inv, w2, b):
    def body(o_ref, h_ref, s_ref, inv_ref, w_ref, b_ref, hn_ref, gn_ref):
        ssum = s_ref[0] + s_ref[1]
        inv_v = inv_ref[...]
        x2 = -2.0 * inv_v * ssum - h_ref[...]
        hn = jnp.maximum(
            o_ref[...]
            + jnp.dot(x2, w_ref[...], preferred_element_type=jnp.float32)
            + b_ref[...], 0.0)
        hn_ref[...] = hn
        gn_ref[...] = inv_v * hn

    return pl.pallas_call(
        body,
        grid=(_G,),
        in_specs=[
            pl.BlockSpec((_BR, _D), lambda i: (i, 0)),
            pl.BlockSpec((_BR, _D), lambda i: (i, 0)),
            pl.BlockSpec((_NC, _BR, _D), lambda i: (0, i, 0)),
            pl.BlockSpec((_BR, _D), lambda i: (i, 0)),
            pl.BlockSpec((_D, _D), lambda i: (0, 0)),
            pl.BlockSpec((1, _D), lambda i: (0, 0)),
        ],
        out_specs=[
            pl.BlockSpec((_BR, _D), lambda i: (i, 0)),
            pl.BlockSpec((_BR, _D), lambda i: (i, 0)),
        ],
        out_shape=[
            jax.ShapeDtypeStruct((_N, _D), jnp.float32),
            jax.ShapeDtypeStruct((_N, _D), jnp.float32),
        ],
    )(out_acc, h, s_part, inv, w2, b)


def _tc_head(h, w0, b0, w1, b1, w2, b2):
    def body(h_ref, w0_ref, b0_ref, w1_ref, b1_ref, w2_ref, b2_ref,
             out_ref, acc_ref):
        i = pl.program_id(0)

        @pl.when(i == 0)
        def _():
            acc_ref[...] = jnp.zeros_like(acc_ref)

        acc_ref[...] += jnp.sum(h_ref[...], axis=0, keepdims=True)

        @pl.when(i == _G - 1)
        def _():
            hg = acc_ref[...] * (1.0 / _N)
            x = jnp.maximum(
                jnp.dot(hg, w0_ref[...], preferred_element_type=jnp.float32)
                + b0_ref[...], 0.0)
            x = jnp.maximum(
                jnp.dot(x, w1_ref[...], preferred_element_type=jnp.float32)
                + b1_ref[...], 0.0)
            out_ref[...] = (
                jnp.dot(x, w2_ref[...], preferred_element_type=jnp.float32)
                + b2_ref[...])

    return pl.pallas_call(
        body,
        grid=(_G,),
        in_specs=[
            pl.BlockSpec((_BR, _D), lambda i: (i, 0)),
            pl.BlockSpec((_D, _D // 2), lambda i: (0, 0)),
            pl.BlockSpec((1, _D // 2), lambda i: (0, 0)),
            pl.BlockSpec((_D // 2, _D // 4), lambda i: (0, 0)),
            pl.BlockSpec((1, _D // 4), lambda i: (0, 0)),
            pl.BlockSpec((_D // 4, 40), lambda i: (0, 0)),
            pl.BlockSpec((1, 40), lambda i: (0, 0)),
        ],
        out_specs=pl.BlockSpec((1, 40), lambda i: (0, 0)),
        out_shape=jax.ShapeDtypeStruct((1, 40), jnp.float32),
        scratch_shapes=[pltpu.VMEM((1, _D), jnp.float32)],
    )(h, w0, b0, w1, b1, w2, b2)


def kernel(features, edge_index, cheb_W, cheb_b,
           mlp_W0, mlp_b0, mlp_W1, mlp_b1, mlp_W2, mlp_b2):
    src = edge_index[0]
    dst = edge_index[1]

    degp = _sc_degree(dst)
    inv, g = _tc_prep(degp, features)

    h = features
    n_layers = cheb_W.shape[0]
    for layer in range(n_layers):
        s1 = _sc_segsum(g, src, dst)
        out_acc, g2 = _tc_layer1(h, s1, inv, cheb_W[layer, 0:2])
        s2 = _sc_segsum(g2, src, dst)
        h, g = _tc_layer2(out_acc, h, s2, inv, cheb_W[layer, 2],
                          cheb_b[layer].reshape(1, -1))

    return _tc_head(h, mlp_W0, mlp_b0.reshape(1, -1),
                    mlp_W1, mlp_b1.reshape(1, -1),
                    mlp_W2, mlp_b2.reshape(1, -1))
